# baseline (speedup 1.0000x reference)
import jax
import jax.numpy as jnp
from jax.experimental import pallas as pl
from jax.experimental.pallas import tpu as pltpu

_DEPTH = 2000
_DBLK = 1000
_KSPLIT = _DEPTH // _DBLK
_NBUF = 4


def _onehot_body(idx_ref, out_hbm, *rest):
    bufs, sems = rest[:_NBUF], rest[_NBUF]
    j = pl.program_id(0)
    k = pl.program_id(1)
    s = j * _KSPLIT + k
    last = pl.num_programs(0) * _KSPLIT - 1
    sl = jax.lax.rem(s, _NBUF)
    idx = idx_ref[0, 0, :]
    d0 = k * _DBLK
    iota = jax.lax.broadcasted_iota(jnp.int32, (_DBLK, idx.shape[0]), 0)
    block = (iota == (idx - d0)[None, :]).astype(jnp.float32)
    dst = out_hbm.at[j, pl.ds(k * _DBLK, _DBLK), :]
    for b in range(_NBUF):

        @pl.when(sl == b)
        def _(b=b):
            @pl.when(s >= _NBUF)
            def _():
                pltpu.make_async_copy(bufs[b], dst, sems.at[b]).wait()

            bufs[b][...] = block
            pltpu.make_async_copy(bufs[b], dst, sems.at[b]).start()

    @pl.when(s == last)
    def _():
        for b in range(_NBUF):
            pltpu.make_async_copy(bufs[b], dst, sems.at[b]).wait()


def kernel(inputs):
    n, m = inputs.shape
    idx_t = inputs.astype(jnp.int32).T.reshape(m, 1, n)
    out = pl.pallas_call(
        _onehot_body,
        grid=(m, _KSPLIT),
        in_specs=[pl.BlockSpec((1, 1, n), lambda j, k: (j, 0, 0))],
        out_specs=pl.BlockSpec(memory_space=pl.ANY),
        out_shape=jax.ShapeDtypeStruct((m, _DEPTH, n), jnp.float32),
        scratch_shapes=(
            [pltpu.VMEM((_DBLK, n), jnp.float32) for _ in range(_NBUF)]
            + [pltpu.SemaphoreType.DMA((_NBUF,))]
        ),
    )(idx_t)
    return out.transpose(2, 0, 1)


# speedup vs baseline: 1.0262x; 1.0262x over previous
import jax
import jax.numpy as jnp
from jax.experimental import pallas as pl

_DEPTH = 2000
_DBLK = 1000


def _onehot_block(idx_ref, out_ref):
    j = pl.program_id(0)
    idx = idx_ref[0, j, :]
    d0 = pl.program_id(1) * _DBLK
    iota = jax.lax.broadcasted_iota(jnp.int32, (_DBLK, idx.shape[0]), 0)
    out_ref[0] = (iota == (idx - d0)[None, :]).astype(jnp.float32)


def kernel(inputs):
    n, m = inputs.shape
    idx_t = inputs.astype(jnp.int32).T.reshape(1, m, n)
    out = pl.pallas_call(
        _onehot_block,
        grid=(m, _DEPTH // _DBLK),
        in_specs=[pl.BlockSpec((1, m, n), lambda j, k: (0, 0, 0))],
        out_specs=pl.BlockSpec((1, _DBLK, n), lambda j, k: (j, k, 0)),
        out_shape=jax.ShapeDtypeStruct((m, _DEPTH, n), jnp.float32),
    )(idx_t)
    return out.transpose(2, 0, 1)
